# baseline (speedup 1.0000x reference)
import functools

import jax
import jax.numpy as jnp
from jax import lax
from jax.experimental import pallas as pl
from jax.experimental.pallas import tpu as pltpu
from jax.experimental.pallas import tpu_sc as plsc

_N_OBJ = 8
_N_P = 64
_P2 = 4096
_C = 256
_H = 128
_W = 128
_IMG = 512.0
_SIM_THR = 0.5
_LOSS_WEIGHT = 1.0

_N_KP = _N_OBJ * _N_P
_P_TOTAL = _N_KP + _N_OBJ * _P2
_NW = 32
_PPW = _P_TOTAL // _NW
_CH = 40
_NCHUNK = _PPW // _CH
_DEPTH = 4


def _make_gatherer():
    mesh = plsc.VectorSubcoreMesh(core_axis_name="c", subcore_axis_name="s")

    @functools.partial(
        pl.kernel,
        mesh=mesh,
        out_type=jax.ShapeDtypeStruct((4, _P_TOTAL, _C // 2), jnp.uint32),
        scratch_types=(
            [pltpu.VMEM((_NCHUNK * 4, _CH), jnp.int32)]
            + [pltpu.VMEM((_CH, _C // 2), jnp.uint32) for _ in range(8)]
            + [pltpu.SemaphoreType.DMA for _ in range(4)]
        ),
    )
    def gatherer(table_hbm, idx_hbm, out_hbm, idx_v,
                 a0, a1, a2, a3, c0, c1, c2, c3, gsa, gsb, osa, osb):
        wid = lax.axis_index("s") * 2 + lax.axis_index("c")
        base = wid * _PPW
        pltpu.sync_copy(idx_hbm.at[wid], idx_v)

        bufs = ((a0, a1, a2, a3), (c0, c1, c2, c3))
        gsems = (gsa, gsb)
        osems = (osa, osb)

        def fire(k, s):
            for c in range(4):
                pltpu.async_copy(
                    table_hbm.at[idx_v.at[k * 4 + c]], bufs[s][c], gsems[s])

        def wait_gather(s):
            for c in range(4):
                pltpu.make_async_copy(
                    table_hbm.at[idx_v.at[c]], bufs[s][c], gsems[s]).wait()

        def fire_out(k, s):
            for c in range(4):
                pltpu.async_copy(
                    bufs[s][c],
                    out_hbm.at[c].at[pl.ds(base + k * _CH, _CH)], osems[s])

        def wait_out(k, s):
            for c in range(4):
                pltpu.make_async_copy(
                    bufs[s][c],
                    out_hbm.at[c].at[pl.ds(base + k * _CH, _CH)],
                    osems[s]).wait()

        fire(0, 0)

        def outer(j, carry):
            for b in range(2):
                k = j * 2 + b
                wait_gather(b)
                fire_out(k, b)

                @pl.when(k + 1 < _NCHUNK)
                def _():
                    @pl.when(k >= 1)
                    def _():
                        wait_out(k - 1, 1 - b)

                    fire(k + 1, 1 - b)
            return carry

        lax.fori_loop(0, _NCHUNK // 2, outer, 0, unroll=False)
        for b in range(2):
            wait_out(_NCHUNK - 2 + b, b)

    return gatherer


@functools.cache
def _get_gatherer():
    return _make_gatherer()


def _unpack_halves(xc):
    ev = lax.bitcast_convert_type(xc << jnp.uint32(16), jnp.float32)
    od = lax.bitcast_convert_type(xc & jnp.uint32(0xFFFF0000), jnp.float32)
    return ev, od


def _loss_body(f1c_ref, f2c_ref, w1_ref, w2_ref,
               x1_ref, y1_ref, x2_ref, y2_ref, out_ref):
    i = pl.program_id(0)
    w1 = w1_ref[...]
    w2 = w2_ref[...]
    hc = _C // 2
    f1e = jnp.zeros((_N_P, hc), jnp.float32)
    f1o = jnp.zeros((_N_P, hc), jnp.float32)
    f2e = jnp.zeros((_P2, hc), jnp.float32)
    f2o = jnp.zeros((_P2, hc), jnp.float32)
    for c in range(4):
        ev, od = _unpack_halves(f1c_ref[c])
        f1e = f1e + w1[:, c][:, None] * ev
        f1o = f1o + w1[:, c][:, None] * od
        ev, od = _unpack_halves(f2c_ref[c])
        f2e = f2e + w2[:, c][:, None] * ev
        f2o = f2o + w2[:, c][:, None] * od

    dn = (((1,), (1,)), ((), ()))
    num = (lax.dot_general(f1e, f2e, dn, preferred_element_type=jnp.float32)
           + lax.dot_general(f1o, f2o, dn,
                             preferred_element_type=jnp.float32))
    na = jnp.sqrt(jnp.sum(f1e * f1e, axis=1)
                  + jnp.sum(f1o * f1o, axis=1))[:, None]
    nb = jnp.sqrt(jnp.sum(f2e * f2e, axis=1)
                  + jnp.sum(f2o * f2o, axis=1))[None, :]
    thr = _SIM_THR * jnp.maximum(na * nb, 1e-8)
    mask = num >= thr

    x1 = x1_ref[0, 0]
    y1 = y1_ref[0, 0]
    x2 = x2_ref[0, 0]
    y2 = y2_ref[0, 0]
    dx = x1[:, None] - x2[None, :]
    dy = y1[:, None] - y2[None, :]
    dist = jnp.sqrt(dx * dx + dy * dy)

    maskf = mask.astype(jnp.float32)
    cnt = jnp.sum(maskf, axis=1)
    d1 = jnp.min(jnp.where(mask, dist, 1e10), axis=1)
    d2 = jnp.sum(dist * maskf, axis=1) / jnp.maximum(cnt, 1.0)
    m = (jnp.sum(x2) + jnp.sum(y2) >= 0).astype(jnp.float32)
    cost = jnp.where(cnt > 0, (d1 + d2) * (0.5 * m), 0.0)
    obj = jnp.sum(cost)

    @pl.when(i == 0)
    def _():
        out_ref[0, 0] = 0.0

    out_ref[0, 0] += obj

    @pl.when(i == _N_OBJ - 1)
    def _():
        out_ref[0, 0] *= _LOSS_WEIGHT / (_N_P * _N_OBJ)


def _loss(corners, wts, x1, y1, x2, y2):
    kp_blk = _N_OBJ * _P2 // _N_P
    out = pl.pallas_call(
        _loss_body,
        grid=(_N_OBJ,),
        in_specs=[
            pl.BlockSpec((4, _N_P, _C // 2), lambda i: (0, kp_blk + i, 0)),
            pl.BlockSpec((4, _P2, _C // 2), lambda i: (0, i, 0)),
            pl.BlockSpec((_N_P, 4), lambda i: (kp_blk + i, 0)),
            pl.BlockSpec((_P2, 4), lambda i: (i, 0)),
            pl.BlockSpec((1, 1, _N_P), lambda i: (i, 0, 0)),
            pl.BlockSpec((1, 1, _N_P), lambda i: (i, 0, 0)),
            pl.BlockSpec((1, 1, _P2), lambda i: (i, 0, 0)),
            pl.BlockSpec((1, 1, _P2), lambda i: (i, 0, 0)),
        ],
        out_specs=pl.BlockSpec((1, 1), lambda i: (0, 0),
                               memory_space=pltpu.SMEM),
        out_shape=jax.ShapeDtypeStruct((1, 1), jnp.float32),
    )(corners, corners, wts, wts, x1, y1, x2, y2)
    return out[0, 0]


def kernel(point_set_1, point_set_2, feats, key_points):
    fbits = lax.bitcast_convert_type(
        feats[0].reshape(_C, _H * _W), jnp.uint32)

    def _rne(u):
        return (u + jnp.uint32(0x7FFF) + ((u >> jnp.uint32(16))
                                          & jnp.uint32(1))) >> jnp.uint32(16)

    packed = (_rne(fbits[1::2]) << jnp.uint32(16)) | _rne(fbits[0::2])
    table = packed.T

    pts = jnp.concatenate(
        [point_set_2.reshape(-1, 2) / _IMG, key_points / _IMG], axis=0)
    ix = pts[:, 0] * _W - 0.5
    iy = pts[:, 1] * _H - 0.5
    x0 = jnp.floor(ix)
    y0 = jnp.floor(iy)
    wx1 = ix - x0
    wx0 = 1.0 - wx1
    wy1 = iy - y0
    wy0 = 1.0 - wy1

    idxs = []
    ws = []
    for xi, yi, wgt in ((x0, y0, wx0 * wy0), (x0 + 1.0, y0, wx1 * wy0),
                        (x0, y0 + 1.0, wx0 * wy1), (x0 + 1.0, y0 + 1.0, wx1 * wy1)):
        valid = ((xi >= 0) & (xi <= _W - 1) & (yi >= 0) & (yi <= _H - 1))
        xc = jnp.clip(xi, 0, _W - 1).astype(jnp.int32)
        yc = jnp.clip(yi, 0, _H - 1).astype(jnp.int32)
        idxs.append(yc * _W + xc)
        ws.append(wgt * valid.astype(jnp.float32))
    idx = (jnp.stack(idxs, axis=0)
           .reshape(4, _NW, _NCHUNK, _CH)
           .transpose(1, 2, 0, 3)
           .reshape(_NW, _NCHUNK * 4, _CH))
    wts = jnp.stack(ws, axis=1)

    corners = _get_gatherer()(table, idx)

    x1 = point_set_1[..., 0].reshape(_N_OBJ, 1, _N_P)
    y1 = point_set_1[..., 1].reshape(_N_OBJ, 1, _N_P)
    x2 = point_set_2[..., 0].reshape(_N_OBJ, 1, _P2)
    y2 = point_set_2[..., 1].reshape(_N_OBJ, 1, _P2)

    return _loss(corners, wts, x1, y1, x2, y2)
